# Initial kernel scaffold; baseline (speedup 1.0000x reference)
import jax
import jax.numpy as jnp
from jax.experimental import pallas as pl


def kernel(x, edge_index, W1, b1, W2, b2):
    raise NotImplementedError("write your pallas kernel here")


# speedup vs baseline: 40.3938x; 40.3938x over previous
import functools

import jax
import jax.numpy as jnp
from jax import lax
from jax.experimental import pallas as pl
from jax.experimental.pallas import tpu as pltpu
from jax.experimental.pallas import tpu_sc as plsc

N = 10000
D_IN = 128
D_H = 8
D_OUT = 128

NSC = 2
TPS = 16
NTILES = NSC * TPS
C = 128
NPAD = 10240
RPT = NPAD // TPS


def _sc_pass(src3, dst3, g, zeros_g, nch):
    mesh = plsc.VectorSubcoreMesh(core_axis_name="c", subcore_axis_name="s")

    @functools.partial(
        pl.kernel,
        mesh=mesh,
        out_type=jax.ShapeDtypeStruct((NSC, NPAD, D_H), jnp.float32),
        scratch_types=[
            pltpu.VMEM((nch, C), jnp.int32),
            pltpu.VMEM((nch, C), jnp.int32),
            pltpu.VMEM((C, D_H), jnp.float32),
            pltpu.SemaphoreType.DMA,
            pltpu.VMEM_SHARED((NPAD, D_H), jnp.float32),
            pltpu.VMEM_SHARED((NPAD, D_H), jnp.float32),
        ],
    )
    def k(src_hbm, dst_hbm, g_hbm, z_hbm, out_hbm,
          src_v, dst_v, msg_v, sem, g_sh, p_sh):
        c = lax.axis_index("c")
        s = lax.axis_index("s")
        wid = c * TPS + s
        rows = pl.ds(s * RPT, RPT)
        pltpu.sync_copy(g_hbm.at[rows], g_sh.at[rows])
        pltpu.sync_copy(z_hbm.at[rows], p_sh.at[rows])
        pltpu.sync_copy(src_hbm.at[wid], src_v)
        pltpu.sync_copy(dst_hbm.at[wid], dst_v)
        plsc.subcore_barrier()

        def chunk(j, carry):
            pltpu.async_copy(g_sh.at[src_v.at[j]], msg_v, sem).wait()
            pltpu.sync_copy(msg_v, p_sh.at[dst_v.at[j]], add=True)
            return carry

        lax.fori_loop(0, nch, chunk, 0)
        plsc.subcore_barrier()
        pltpu.sync_copy(p_sh.at[rows], out_hbm.at[c, rows])

    return k(src3, dst3, g, zeros_g)


def _tc_prep(x, W1, deg2):
    BR = 1000

    def body(x_ref, w_ref, d_ref, g1_ref, dinv_ref):
        deg = d_ref[0] + d_ref[1]
        dinv = lax.rsqrt(deg)
        h = jnp.dot(x_ref[...], w_ref[...], preferred_element_type=jnp.float32)
        g1_ref[...] = h * dinv
        dinv_ref[...] = dinv

    return pl.pallas_call(
        body,
        grid=(N // BR,),
        in_specs=[
            pl.BlockSpec((BR, D_IN), lambda i: (i, 0)),
            pl.BlockSpec((D_IN, D_H), lambda i: (0, 0)),
            pl.BlockSpec((NSC, BR, D_H), lambda i: (0, i, 0)),
        ],
        out_specs=[
            pl.BlockSpec((BR, D_H), lambda i: (i, 0)),
            pl.BlockSpec((BR, D_H), lambda i: (i, 0)),
        ],
        out_shape=[
            jax.ShapeDtypeStruct((N, D_H), jnp.float32),
            jax.ShapeDtypeStruct((N, D_H), jnp.float32),
        ],
    )(x, W1, deg2)


def _tc_mid(p2, dinv, b1):
    BR = 1000

    def body(p_ref, dinv_ref, b_ref, g2_ref):
        p = p_ref[0] + p_ref[1]
        dinv = dinv_ref[...]
        a = jnp.maximum(p * dinv + b_ref[...], 0.0)
        g2_ref[...] = a * dinv

    return pl.pallas_call(
        body,
        grid=(N // BR,),
        in_specs=[
            pl.BlockSpec((NSC, BR, D_H), lambda i: (0, i, 0)),
            pl.BlockSpec((BR, D_H), lambda i: (i, 0)),
            pl.BlockSpec((1, D_H), lambda i: (0, 0)),
        ],
        out_specs=pl.BlockSpec((BR, D_H), lambda i: (i, 0)),
        out_shape=jax.ShapeDtypeStruct((N, D_H), jnp.float32),
    )(p2, dinv, b1)


def _tc_out(p2, dinv, W2, b2):
    BR = 1000

    def body(p_ref, dinv_ref, w_ref, b_ref, out_ref):
        a = (p_ref[0] + p_ref[1]) * dinv_ref[...]
        out_ref[...] = (
            jnp.dot(a, w_ref[...], preferred_element_type=jnp.float32)
            + b_ref[...]
        )

    return pl.pallas_call(
        body,
        grid=(N // BR,),
        in_specs=[
            pl.BlockSpec((NSC, BR, D_H), lambda i: (0, i, 0)),
            pl.BlockSpec((BR, D_H), lambda i: (i, 0)),
            pl.BlockSpec((D_H, D_OUT), lambda i: (0, 0)),
            pl.BlockSpec((1, D_OUT), lambda i: (0, 0)),
        ],
        out_specs=pl.BlockSpec((BR, D_OUT), lambda i: (i, 0)),
        out_shape=jax.ShapeDtypeStruct((N, D_OUT), jnp.float32),
    )(p2, dinv, W2, b2)


def kernel(x, edge_index, W1, b1, W2, b2):
    e = edge_index.shape[1]
    ntot = e + N
    nch = -(-ntot // (NTILES * C))
    epad = NTILES * C * nch - ntot

    ei = edge_index.astype(jnp.int32)
    loop = jnp.arange(N, dtype=jnp.int32)
    padv = jnp.full((epad,), N, jnp.int32)
    src3 = jnp.concatenate([ei[0], loop, padv]).reshape(NTILES, nch, C)
    dst3 = jnp.concatenate([ei[1], loop, padv]).reshape(NTILES, nch, C)

    ones_g = jnp.ones((NPAD, D_H), jnp.float32)
    zeros_g = jnp.zeros((NPAD, D_H), jnp.float32)
    zpad = jnp.zeros((NPAD - N, D_H), jnp.float32)

    deg2 = _sc_pass(src3, dst3, ones_g, zeros_g, nch)
    g1, dinv = _tc_prep(x, W1, deg2[:, :N])
    p1 = _sc_pass(src3, dst3, jnp.concatenate([g1, zpad]), zeros_g, nch)
    g2 = _tc_mid(p1[:, :N], dinv, b1.reshape(1, D_H))
    p2 = _sc_pass(src3, dst3, jnp.concatenate([g2, zpad]), zeros_g, nch)
    return _tc_out(p2[:, :N], dinv, W2, b2.reshape(1, D_OUT))


# speedup vs baseline: 44.9719x; 1.1133x over previous
import functools

import jax
import jax.numpy as jnp
from jax import lax
from jax.experimental import pallas as pl
from jax.experimental.pallas import tpu as pltpu
from jax.experimental.pallas import tpu_sc as plsc

N = 10000
D_IN = 128
D_H = 8
D_OUT = 128

NSC = 2
TPS = 16
NTILES = NSC * TPS
C = 128
NPAD = 10240
RPT = NPAD // TPS


def _sc_pass(src3, dst3, g, zeros_g, nch):
    mesh = plsc.VectorSubcoreMesh(core_axis_name="c", subcore_axis_name="s")

    @functools.partial(
        pl.kernel,
        mesh=mesh,
        out_type=jax.ShapeDtypeStruct((NSC, NPAD, D_H), jnp.float32),
        scratch_types=[
            pltpu.VMEM((nch, C), jnp.int32),
            pltpu.VMEM((nch, C), jnp.int32),
            pltpu.VMEM((C, D_H), jnp.float32),
            pltpu.SemaphoreType.DMA,
            pltpu.VMEM_SHARED((NPAD, D_H), jnp.float32),
            pltpu.VMEM_SHARED((NPAD, D_H), jnp.float32),
        ],
    )
    def k(src_hbm, dst_hbm, g_hbm, z_hbm, out_hbm,
          src_v, dst_v, msg_v, sem, g_sh, p_sh):
        c = lax.axis_index("c")
        s = lax.axis_index("s")
        wid = c * TPS + s
        rows = pl.ds(s * RPT, RPT)
        pltpu.sync_copy(g_hbm.at[rows], g_sh.at[rows])
        pltpu.sync_copy(z_hbm.at[rows], p_sh.at[rows])
        pltpu.sync_copy(src_hbm.at[wid], src_v)
        pltpu.sync_copy(dst_hbm.at[wid], dst_v)
        plsc.subcore_barrier()

        def chunk(j, carry):
            pltpu.async_copy(g_sh.at[src_v.at[j]], msg_v, sem).wait()
            pltpu.sync_copy(msg_v, p_sh.at[dst_v.at[j]], add=True)
            return carry

        lax.fori_loop(0, nch, chunk, 0)
        plsc.subcore_barrier()
        pltpu.sync_copy(p_sh.at[rows], out_hbm.at[c, rows])

    return k(src3, dst3, g, zeros_g)


def _sc_deg(dst3, ones_g, zeros_g, nch):
    mesh = plsc.VectorSubcoreMesh(core_axis_name="c", subcore_axis_name="s")

    @functools.partial(
        pl.kernel,
        mesh=mesh,
        out_type=jax.ShapeDtypeStruct((NSC, NPAD, D_H), jnp.float32),
        scratch_types=[
            pltpu.VMEM((nch, C), jnp.int32),
            pltpu.VMEM((C, D_H), jnp.float32),
            pltpu.VMEM_SHARED((NPAD, D_H), jnp.float32),
        ],
    )
    def k(dst_hbm, ones_hbm, z_hbm, out_hbm, dst_v, msg_v, p_sh):
        c = lax.axis_index("c")
        s = lax.axis_index("s")
        wid = c * TPS + s
        rows = pl.ds(s * RPT, RPT)
        pltpu.sync_copy(z_hbm.at[rows], p_sh.at[rows])
        pltpu.sync_copy(ones_hbm.at[pl.ds(0, C)], msg_v)
        pltpu.sync_copy(dst_hbm.at[wid], dst_v)
        plsc.subcore_barrier()

        def chunk(j, carry):
            pltpu.sync_copy(msg_v, p_sh.at[dst_v.at[j]], add=True)
            return carry

        lax.fori_loop(0, nch, chunk, 0)
        plsc.subcore_barrier()
        pltpu.sync_copy(p_sh.at[rows], out_hbm.at[c, rows])

    return k(dst3, ones_g, zeros_g)


def _tc_prep(x, W1, deg2):
    BR = 1000

    def body(x_ref, w_ref, d_ref, g1_ref, dinv_ref):
        deg = d_ref[0] + d_ref[1]
        dinv = lax.rsqrt(deg)
        h = jnp.dot(x_ref[...], w_ref[...], preferred_element_type=jnp.float32)
        g1_ref[...] = h * dinv
        dinv_ref[...] = dinv

    return pl.pallas_call(
        body,
        grid=(N // BR,),
        in_specs=[
            pl.BlockSpec((BR, D_IN), lambda i: (i, 0)),
            pl.BlockSpec((D_IN, D_H), lambda i: (0, 0)),
            pl.BlockSpec((NSC, BR, D_H), lambda i: (0, i, 0)),
        ],
        out_specs=[
            pl.BlockSpec((BR, D_H), lambda i: (i, 0)),
            pl.BlockSpec((BR, D_H), lambda i: (i, 0)),
        ],
        out_shape=[
            jax.ShapeDtypeStruct((N, D_H), jnp.float32),
            jax.ShapeDtypeStruct((N, D_H), jnp.float32),
        ],
    )(x, W1, deg2)


def _tc_mid(p2, dinv, b1):
    BR = 1000

    def body(p_ref, dinv_ref, b_ref, g2_ref):
        p = p_ref[0] + p_ref[1]
        dinv = dinv_ref[...]
        a = jnp.maximum(p * dinv + b_ref[...], 0.0)
        g2_ref[...] = a * dinv

    return pl.pallas_call(
        body,
        grid=(N // BR,),
        in_specs=[
            pl.BlockSpec((NSC, BR, D_H), lambda i: (0, i, 0)),
            pl.BlockSpec((BR, D_H), lambda i: (i, 0)),
            pl.BlockSpec((1, D_H), lambda i: (0, 0)),
        ],
        out_specs=pl.BlockSpec((BR, D_H), lambda i: (i, 0)),
        out_shape=jax.ShapeDtypeStruct((N, D_H), jnp.float32),
    )(p2, dinv, b1)


def _tc_out(p2, dinv, W2, b2):
    BR = 1000

    def body(p_ref, dinv_ref, w_ref, b_ref, out_ref):
        a = (p_ref[0] + p_ref[1]) * dinv_ref[...]
        out_ref[...] = (
            jnp.dot(a, w_ref[...], preferred_element_type=jnp.float32)
            + b_ref[...]
        )

    return pl.pallas_call(
        body,
        grid=(N // BR,),
        in_specs=[
            pl.BlockSpec((NSC, BR, D_H), lambda i: (0, i, 0)),
            pl.BlockSpec((BR, D_H), lambda i: (i, 0)),
            pl.BlockSpec((D_H, D_OUT), lambda i: (0, 0)),
            pl.BlockSpec((1, D_OUT), lambda i: (0, 0)),
        ],
        out_specs=pl.BlockSpec((BR, D_OUT), lambda i: (i, 0)),
        out_shape=jax.ShapeDtypeStruct((N, D_OUT), jnp.float32),
    )(p2, dinv, W2, b2)


def kernel(x, edge_index, W1, b1, W2, b2):
    e = edge_index.shape[1]
    ntot = e + N
    nch = -(-ntot // (NTILES * C))
    epad = NTILES * C * nch - ntot

    ei = edge_index.astype(jnp.int32)
    loop = jnp.arange(N, dtype=jnp.int32)
    padv = jnp.full((epad,), N, jnp.int32)
    src3 = jnp.concatenate([ei[0], loop, padv]).reshape(NTILES, nch, C)
    dst3 = jnp.concatenate([ei[1], loop, padv]).reshape(NTILES, nch, C)

    ones_g = jnp.ones((NPAD, D_H), jnp.float32)
    zeros_g = jnp.zeros((NPAD, D_H), jnp.float32)
    zpad = jnp.zeros((NPAD - N, D_H), jnp.float32)

    deg2 = _sc_deg(dst3, ones_g, zeros_g, nch)
    g1, dinv = _tc_prep(x, W1, deg2[:, :N])
    p1 = _sc_pass(src3, dst3, jnp.concatenate([g1, zpad]), zeros_g, nch)
    g2 = _tc_mid(p1[:, :N], dinv, b1.reshape(1, D_H))
    p2 = _sc_pass(src3, dst3, jnp.concatenate([g2, zpad]), zeros_g, nch)
    return _tc_out(p2[:, :N], dinv, W2, b2.reshape(1, D_OUT))


# speedup vs baseline: 52.1034x; 1.1586x over previous
import functools

import jax
import jax.numpy as jnp
from jax import lax
from jax.experimental import pallas as pl
from jax.experimental.pallas import tpu as pltpu
from jax.experimental.pallas import tpu_sc as plsc

N = 10000
D_IN = 128
D_H = 8
D_OUT = 128

NSC = 2
TPS = 16
NTILES = NSC * TPS
C = 128
NPAD = 10240
RPT = NPAD // TPS


def _sc_pass(src3, dst3, g, zeros_g, nch):
    mesh = plsc.VectorSubcoreMesh(core_axis_name="c", subcore_axis_name="s")

    @functools.partial(
        pl.kernel,
        mesh=mesh,
        out_type=jax.ShapeDtypeStruct((NSC, NPAD, D_H), jnp.float32),
        scratch_types=[
            pltpu.VMEM((nch, C), jnp.int32),
            pltpu.VMEM((nch, C), jnp.int32),
            pltpu.VMEM((C, D_H), jnp.float32),
            pltpu.SemaphoreType.DMA,
            pltpu.VMEM_SHARED((NPAD, D_H), jnp.float32),
            pltpu.VMEM_SHARED((NPAD, D_H), jnp.float32),
        ],
        compiler_params=pltpu.CompilerParams(use_tc_tiling_on_sc=True),
    )
    def k(src_hbm, dst_hbm, g_hbm, z_hbm, out_hbm,
          src_v, dst_v, msg_v, sem, g_sh, p_sh):
        c = lax.axis_index("c")
        s = lax.axis_index("s")
        wid = c * TPS + s
        rows = pl.ds(s * RPT, RPT)
        @pl.when(s < TPS - 1)
        def _():
            pltpu.sync_copy(g_hbm.at[rows], g_sh.at[rows])

        @pl.when(s == TPS - 1)
        def _():
            last = pl.ds((TPS - 1) * RPT, N - (TPS - 1) * RPT)
            pltpu.sync_copy(g_hbm.at[last], g_sh.at[last])

        pltpu.sync_copy(z_hbm.at[rows], p_sh.at[rows])
        pltpu.sync_copy(src_hbm.at[wid], src_v)
        pltpu.sync_copy(dst_hbm.at[wid], dst_v)
        plsc.subcore_barrier()

        def chunk(j, carry):
            pltpu.async_copy(g_sh.at[src_v.at[j]], msg_v, sem).wait()
            pltpu.sync_copy(msg_v, p_sh.at[dst_v.at[j]], add=True)
            return carry

        lax.fori_loop(0, nch, chunk, 0)
        plsc.subcore_barrier()
        pltpu.sync_copy(p_sh.at[rows], out_hbm.at[c, rows])

    return k(src3, dst3, g, zeros_g)


def _sc_deg(dst3, ones_g, zeros_g, nch):
    mesh = plsc.VectorSubcoreMesh(core_axis_name="c", subcore_axis_name="s")

    @functools.partial(
        pl.kernel,
        mesh=mesh,
        out_type=jax.ShapeDtypeStruct((NSC, NPAD, D_H), jnp.float32),
        scratch_types=[
            pltpu.VMEM((nch, C), jnp.int32),
            pltpu.VMEM((C, D_H), jnp.float32),
            pltpu.VMEM_SHARED((NPAD, D_H), jnp.float32),
        ],
        compiler_params=pltpu.CompilerParams(use_tc_tiling_on_sc=True),
    )
    def k(dst_hbm, ones_hbm, z_hbm, out_hbm, dst_v, msg_v, p_sh):
        c = lax.axis_index("c")
        s = lax.axis_index("s")
        wid = c * TPS + s
        rows = pl.ds(s * RPT, RPT)
        pltpu.sync_copy(z_hbm.at[rows], p_sh.at[rows])
        pltpu.sync_copy(ones_hbm.at[pl.ds(0, C)], msg_v)
        pltpu.sync_copy(dst_hbm.at[wid], dst_v)
        plsc.subcore_barrier()

        def chunk(j, carry):
            pltpu.sync_copy(msg_v, p_sh.at[dst_v.at[j]], add=True)
            return carry

        lax.fori_loop(0, nch, chunk, 0)
        plsc.subcore_barrier()
        pltpu.sync_copy(p_sh.at[rows], out_hbm.at[c, rows])

    return k(dst3, ones_g, zeros_g)


def _tc_prep(x, W1, deg2):
    BR = 1000

    def body(x_ref, w_ref, d_ref, g1_ref, dinv_ref):
        deg = d_ref[0] + d_ref[1]
        dinv = lax.rsqrt(deg)
        h = jnp.dot(x_ref[...], w_ref[...], preferred_element_type=jnp.float32)
        g1_ref[...] = h * dinv
        dinv_ref[...] = dinv

    return pl.pallas_call(
        body,
        grid=(N // BR,),
        in_specs=[
            pl.BlockSpec((BR, D_IN), lambda i: (i, 0)),
            pl.BlockSpec((D_IN, D_H), lambda i: (0, 0)),
            pl.BlockSpec((NSC, BR, D_H), lambda i: (0, i, 0)),
        ],
        out_specs=[
            pl.BlockSpec((BR, D_H), lambda i: (i, 0)),
            pl.BlockSpec((BR, D_H), lambda i: (i, 0)),
        ],
        out_shape=[
            jax.ShapeDtypeStruct((N, D_H), jnp.float32),
            jax.ShapeDtypeStruct((N, D_H), jnp.float32),
        ],
    )(x, W1, deg2)


def _tc_mid(p2, dinv, b1):
    BR = 1000

    def body(p_ref, dinv_ref, b_ref, g2_ref):
        p = p_ref[0] + p_ref[1]
        dinv = dinv_ref[...]
        a = jnp.maximum(p * dinv + b_ref[...], 0.0)
        g2_ref[...] = a * dinv

    return pl.pallas_call(
        body,
        grid=(N // BR,),
        in_specs=[
            pl.BlockSpec((NSC, BR, D_H), lambda i: (0, i, 0)),
            pl.BlockSpec((BR, D_H), lambda i: (i, 0)),
            pl.BlockSpec((1, D_H), lambda i: (0, 0)),
        ],
        out_specs=pl.BlockSpec((BR, D_H), lambda i: (i, 0)),
        out_shape=jax.ShapeDtypeStruct((N, D_H), jnp.float32),
    )(p2, dinv, b1)


def _tc_out(p2, dinv, W2, b2):
    BR = 1000

    def body(p_ref, dinv_ref, w_ref, b_ref, out_ref):
        a = (p_ref[0] + p_ref[1]) * dinv_ref[...]
        out_ref[...] = (
            jnp.dot(a, w_ref[...], preferred_element_type=jnp.float32)
            + b_ref[...]
        )

    return pl.pallas_call(
        body,
        grid=(N // BR,),
        in_specs=[
            pl.BlockSpec((NSC, BR, D_H), lambda i: (0, i, 0)),
            pl.BlockSpec((BR, D_H), lambda i: (i, 0)),
            pl.BlockSpec((D_H, D_OUT), lambda i: (0, 0)),
            pl.BlockSpec((1, D_OUT), lambda i: (0, 0)),
        ],
        out_specs=pl.BlockSpec((BR, D_OUT), lambda i: (i, 0)),
        out_shape=jax.ShapeDtypeStruct((N, D_OUT), jnp.float32),
    )(p2, dinv, W2, b2)


def kernel(x, edge_index, W1, b1, W2, b2):
    e = edge_index.shape[1]
    ntot = e + N
    nch = -(-ntot // (NTILES * C))
    epad = NTILES * C * nch - ntot

    ei = edge_index.astype(jnp.int32)
    loop = jnp.arange(N, dtype=jnp.int32)
    padv = jnp.full((epad,), N, jnp.int32)
    src3 = jnp.concatenate([ei[0], loop, padv]).reshape(NTILES, nch, C)
    dst3 = jnp.concatenate([ei[1], loop, padv]).reshape(NTILES, nch, C)

    ones_g = jnp.ones((NPAD, D_H), jnp.float32)
    zeros_g = jnp.zeros((NPAD, D_H), jnp.float32)

    deg2 = _sc_deg(dst3, ones_g, zeros_g, nch)
    g1, dinv = _tc_prep(x, W1, deg2)
    p1 = _sc_pass(src3, dst3, g1, zeros_g, nch)
    g2 = _tc_mid(p1, dinv, b1.reshape(1, D_H))
    p2 = _sc_pass(src3, dst3, g2, zeros_g, nch)
    return _tc_out(p2, dinv, W2, b2.reshape(1, D_OUT))
